# baseline (speedup 1.0000x reference)
import functools

import jax
import jax.numpy as jnp
from jax.experimental import pallas as pl
from jax.experimental.pallas import tpu as pltpu

_N_KF = 512
_K = 20
_NROWS = 2 * _K + 2


def _select_kernel(locs_t_ref, last_ref, idx_ref):
    diff = locs_t_ref[...] - last_ref[...]
    dist = jnp.sqrt(jnp.sum(diff * diff, axis=0, keepdims=True))
    iota = jax.lax.broadcasted_iota(jnp.int32, (1, _N_KF), 1)
    iota_out = jax.lax.broadcasted_iota(jnp.int32, (1, 32), 1)
    acc = jnp.zeros((1, 32), jnp.int32)
    prev_v = jnp.float32(-jnp.inf)
    prev_i = jnp.int32(-1)
    for i in range(_K):
        elig = (dist > prev_v) | ((dist == prev_v) & (iota > prev_i))
        v = jnp.where(elig, dist, jnp.inf)
        m = jnp.min(v)
        sel = jnp.min(jnp.where(v == m, iota, _N_KF))
        acc = jnp.where(iota_out == i, sel, acc)
        prev_v, prev_i = m, sel
    idx_ref[...] = acc


def _gather_kernel(kf_i_ref, nf_i_ref, locs_ref, kfk_ref, kfd_ref,
                   nfk_ref, nfd_ref, kpts_out, descs_out, mask_out, locs_out):
    row = pl.program_id(0)

    @pl.when(row < 2 * _K)
    def _():
        kpts_out[...] = kfk_ref[...]
        descs_out[...] = kfd_ref[...]
        locs_out[...] = locs_ref[...]

    @pl.when(row >= 2 * _K)
    def _():
        kpts_out[...] = nfk_ref[...]
        descs_out[...] = nfd_ref[...]

    kpts = kpts_out[0]
    descs = descs_out[0]
    fin_k = jnp.all(jnp.isfinite(kpts), axis=-1)
    fin_d = jnp.all(jnp.isfinite(descs), axis=-1)
    nk = jnp.sqrt(jnp.sum(kpts * kpts, axis=-1))
    nd = jnp.sqrt(jnp.sum(descs * descs, axis=-1))
    m = fin_k & fin_d & (nk >= 1e-6) & (nd >= 1e-6)
    mask_out[0, 0, :] = m.astype(jnp.int32)


def kernel(keyframe_locs, keyframe_kpts, keyframe_descs, last_loc,
           newframe_kpts, newframe_descs):
    n_kf, f = keyframe_kpts.shape[0], keyframe_kpts.shape[1]
    d = keyframe_descs.shape[2]
    b = newframe_kpts.shape[0]

    locs_t = keyframe_locs.T
    last_c = last_loc.reshape(3, 1)
    topk32 = pl.pallas_call(
        _select_kernel,
        out_shape=jax.ShapeDtypeStruct((1, 32), jnp.int32),
    )(locs_t, last_c)
    topk = topk32.reshape(32)[:_K]

    temporal = jnp.arange(n_kf - _K, n_kf, dtype=jnp.int32)
    kf_i = jnp.concatenate([temporal, topk, topk[-1:], topk[-1:]])
    nf_i = jnp.concatenate([jnp.zeros((2 * _K,), jnp.int32),
                            jnp.arange(b, dtype=jnp.int32)])

    locs3 = keyframe_locs.reshape(n_kf, 1, 3)

    grid_spec = pltpu.PrefetchScalarGridSpec(
        num_scalar_prefetch=2,
        grid=(_NROWS,),
        in_specs=[
            pl.BlockSpec((1, 1, 3), lambda i, kf, nf: (kf[i], 0, 0)),
            pl.BlockSpec((1, f, 3), lambda i, kf, nf: (kf[i], 0, 0)),
            pl.BlockSpec((1, f, d), lambda i, kf, nf: (kf[i], 0, 0)),
            pl.BlockSpec((1, f, 3), lambda i, kf, nf: (nf[i], 0, 0)),
            pl.BlockSpec((1, f, d), lambda i, kf, nf: (nf[i], 0, 0)),
        ],
        out_specs=[
            pl.BlockSpec((1, f, 3), lambda i, kf, nf: (i, 0, 0)),
            pl.BlockSpec((1, f, d), lambda i, kf, nf: (i, 0, 0)),
            pl.BlockSpec((1, 1, f), lambda i, kf, nf: (i, 0, 0)),
            pl.BlockSpec((1, 1, 3),
                         lambda i, kf, nf: (jnp.minimum(i, 2 * _K - 1), 0, 0)),
        ],
    )
    kpts, descs, mask_i, locs_sel = pl.pallas_call(
        _gather_kernel,
        grid_spec=grid_spec,
        out_shape=[
            jax.ShapeDtypeStruct((_NROWS, f, 3), jnp.float32),
            jax.ShapeDtypeStruct((_NROWS, f, d), jnp.float32),
            jax.ShapeDtypeStruct((_NROWS, 1, f), jnp.int32),
            jax.ShapeDtypeStruct((2 * _K, 1, 3), jnp.float32),
        ],
    )(kf_i, nf_i, locs3, keyframe_kpts, keyframe_descs,
      newframe_kpts, newframe_descs)

    curr_mask = mask_i.reshape(_NROWS, f).astype(bool)
    kf_locs = locs_sel.reshape(2 * _K, 3)
    return (kpts, descs, curr_mask, kf_locs)


# speedup vs baseline: 3.0956x; 3.0956x over previous
import functools

import jax
import jax.numpy as jnp
from jax.experimental import pallas as pl
from jax.experimental.pallas import tpu as pltpu

_N_KF = 512
_K = 20
_NROWS = 2 * _K + 2
_PREC = jax.lax.Precision.HIGHEST


def _select_kernel(locs_t_ref, last_ref, idx_ref, locs_out_ref):
    diff = locs_t_ref[...] - last_ref[...]
    d = jnp.sqrt(jnp.sum(diff * diff, axis=0, keepdims=True))
    dcol = d.reshape(_N_KF, 1)
    irow = jax.lax.broadcasted_iota(jnp.int32, (1, _N_KF), 1)
    icol = jax.lax.broadcasted_iota(jnp.int32, (_N_KF, 1), 0)
    smaller = (d < dcol) | ((d == dcol) & (irow < icol))
    rank = jnp.sum(smaller.astype(jnp.int32), axis=1, keepdims=True)
    j64 = jax.lax.broadcasted_iota(jnp.int32, (1, 64), 1)
    onehot = (rank == (j64 - _K)).astype(jnp.float32)
    idxf = jax.lax.dot_general(
        irow.astype(jnp.float32), onehot,
        (((1,), (0,)), ((), ())), precision=_PREC)
    idx = jnp.where(j64 < _K, (_N_KF - _K) + j64,
                    idxf.astype(jnp.int32))
    idx_ref[...] = idx
    sel = (icol == idx).astype(jnp.float32)
    locs_out_ref[...] = jax.lax.dot_general(
        locs_t_ref[...], sel, (((1,), (0,)), ((), ())), precision=_PREC)


def _gather_kernel(kf_i_ref, nf_i_ref, kfk_ref, kfd_ref,
                   nfk_ref, nfd_ref, kpts_out, descs_out, mask_out):
    row = pl.program_id(0)

    @pl.when(row < 2 * _K)
    def _():
        kpts_out[...] = kfk_ref[...]
        descs_out[...] = kfd_ref[...]

    @pl.when(row >= 2 * _K)
    def _():
        kpts_out[...] = nfk_ref[...]
        descs_out[...] = nfd_ref[...]

    kpts = kpts_out[0]
    descs = descs_out[0]
    nk2 = jnp.sum(kpts * kpts, axis=0, keepdims=True)
    dsq = descs * descs
    ones = jnp.ones((1, descs.shape[1]), jnp.float32)
    nd2 = jax.lax.dot_general(
        ones, dsq, (((1,), (1,)), ((), ())), precision=_PREC)
    m = (jnp.isfinite(nk2) & jnp.isfinite(nd2)
         & (jnp.sqrt(nk2) >= 1e-6) & (jnp.sqrt(nd2) >= 1e-6))
    mask_out[0] = m.astype(jnp.int32)


def kernel(keyframe_locs, keyframe_kpts, keyframe_descs, last_loc,
           newframe_kpts, newframe_descs):
    n_kf, f = keyframe_kpts.shape[0], keyframe_kpts.shape[1]
    d = keyframe_descs.shape[2]
    b = newframe_kpts.shape[0]

    locs_t = keyframe_locs.T
    last_c = last_loc.reshape(3, 1)
    idx64, locs_sel = pl.pallas_call(
        _select_kernel,
        out_shape=[jax.ShapeDtypeStruct((1, 64), jnp.int32),
                   jax.ShapeDtypeStruct((3, 64), jnp.float32)],
    )(locs_t, last_c)
    kf_locs = locs_sel[:, :2 * _K].T

    src40 = idx64.reshape(64)[:2 * _K]
    kf_i = jnp.concatenate([src40, src40[-1:], src40[-1:]])
    nf_i = jnp.concatenate([jnp.zeros((2 * _K,), jnp.int32),
                            jnp.arange(b, dtype=jnp.int32)])

    kf_kpts_t = jnp.swapaxes(keyframe_kpts, 1, 2)
    nf_kpts_t = jnp.swapaxes(newframe_kpts, 1, 2)

    grid_spec = pltpu.PrefetchScalarGridSpec(
        num_scalar_prefetch=2,
        grid=(_NROWS,),
        in_specs=[
            pl.BlockSpec((1, 3, f), lambda i, kf, nf: (kf[i], 0, 0)),
            pl.BlockSpec((1, f, d), lambda i, kf, nf: (kf[i], 0, 0)),
            pl.BlockSpec((1, 3, f), lambda i, kf, nf: (nf[i], 0, 0)),
            pl.BlockSpec((1, f, d), lambda i, kf, nf: (nf[i], 0, 0)),
        ],
        out_specs=[
            pl.BlockSpec((1, 3, f), lambda i, kf, nf: (i, 0, 0)),
            pl.BlockSpec((1, f, d), lambda i, kf, nf: (i, 0, 0)),
            pl.BlockSpec((1, 1, f), lambda i, kf, nf: (i, 0, 0)),
        ],
    )
    kpts_t, descs, mask_i = pl.pallas_call(
        _gather_kernel,
        grid_spec=grid_spec,
        out_shape=[
            jax.ShapeDtypeStruct((_NROWS, 3, f), jnp.float32),
            jax.ShapeDtypeStruct((_NROWS, f, d), jnp.float32),
            jax.ShapeDtypeStruct((_NROWS, 1, f), jnp.int32),
        ],
    )(kf_i, nf_i, kf_kpts_t, keyframe_descs, nf_kpts_t, newframe_descs)

    curr_kpts = jnp.swapaxes(kpts_t, 1, 2)
    curr_mask = mask_i.reshape(_NROWS, f).astype(bool)
    return (curr_kpts, descs, curr_mask, kf_locs)
